# baseline (speedup 1.0000x reference)
import jax
import jax.numpy as jnp
from jax import lax
from jax.experimental import pallas as pl
from jax.experimental.pallas import tpu as pltpu
from jax.experimental.pallas import tpu_sc as plsc

_K = 8
_ROWS = 256
_DPAD = 8
_B = 4
_N = 4096
_SPLIT = 3328
_NW = 32
_L = 16


def _tc_block(q_ref, kt_ref, out_ref):
    b = pl.program_id(0)
    i = pl.program_id(1)

    q = q_ref[0]
    kt = kt_ref[0]

    sq_q = jnp.sum(q * q, axis=1, keepdims=True)
    sq_k = jnp.sum(kt * kt, axis=0, keepdims=True)
    mm = jnp.dot(q * -2.0, kt, preferred_element_type=jnp.float32)
    d2 = mm + sq_q + sq_k
    work = d2.astype(jnp.bfloat16)

    s = jnp.zeros((_ROWS,), dtype=jnp.float32)
    for t in range(_K + 1):
        fold = work
        while fold.shape[1] > 128:
            h = fold.shape[1] // 2
            fold = jnp.minimum(fold[:, :h], fold[:, h:])
        mf = jnp.min(fold.astype(jnp.float32), axis=1)
        if t > 0:
            s = s + jnp.where(jnp.isfinite(mf), mf, 0.0)
        if t < _K:
            m = mf.astype(jnp.bfloat16)
            work = jnp.where(work == m[:, None], jnp.bfloat16(jnp.inf), work)

    partial = jnp.sum(s).reshape(1, 1)

    @pl.when((b == 0) & (i == 0))
    def _init():
        out_ref[:, :] = jnp.zeros((1, 1), dtype=jnp.float32)

    out_ref[:, :] += partial


def _tc_part(qp, kt):
    return pl.pallas_call(
        _tc_block,
        grid=(_B, _SPLIT // _ROWS),
        in_specs=[
            pl.BlockSpec((1, _ROWS, _DPAD), lambda b, i: (b, i, 0)),
            pl.BlockSpec((1, _DPAD, _N), lambda b, i: (b, 0, 0)),
        ],
        out_specs=pl.BlockSpec((1, 1), lambda b, i: (0, 0)),
        out_shape=jax.ShapeDtypeStruct((1, 1), jnp.float32),
    )(qp, kt)


def _sc_body(px_hbm, py_hbm, pz_hbm, out_hbm, px_v, py_v, pz_v, obuf):
    rows_per_w = (_N - _SPLIT) // _NW
    wid = lax.axis_index("s") * 2 + lax.axis_index("c")

    total = jnp.zeros((_L,), jnp.float32)
    for b in range(_B):
        pltpu.sync_copy(px_hbm.at[pl.ds(b * _N, _N)], px_v)
        pltpu.sync_copy(py_hbm.at[pl.ds(b * _N, _N)], py_v)
        pltpu.sync_copy(pz_hbm.at[pl.ds(b * _N, _N)], pz_v)

        def row_body(r, tot):
            row = _SPLIT + wid * rows_per_w + r
            base = (row // _L) * _L
            off = jnp.full((_L, 1), row - base, jnp.int32)
            qsl = pl.ds(base, _L)
            dn = lax.GatherDimensionNumbers(
                offset_dims=(), collapsed_slice_dims=(0,),
                start_index_map=(0,))

            def splat(vec):
                return lax.gather(
                    vec, off, dn, slice_sizes=(1,),
                    mode=lax.GatherScatterMode.PROMISE_IN_BOUNDS)

            qx = splat(px_v[qsl])
            qy = splat(py_v[qsl])
            qz = splat(pz_v[qsl])

            inf16 = jnp.full((_L,), jnp.inf, jnp.float32)
            tops0 = (inf16,) * (_K + 1)

            def chunk_body(c, tops):
                sl = pl.ds(c * _L, _L)
                dx = px_v[sl] - qx
                dy = py_v[sl] - qy
                dz = pz_v[sl] - qz
                x = dx * dx + dy * dy + dz * dz
                new = []
                for j in range(_K + 1):
                    lo = jnp.minimum(tops[j], x)
                    x = jnp.maximum(tops[j], x)
                    new.append(lo)
                return tuple(new)

            tops = lax.fori_loop(0, _N // _L, chunk_body, tops0)

            lanes = lax.iota(jnp.int32, _L)

            def xmin_splat(v):
                for kk in (8, 4, 2, 1):
                    idx = (lanes ^ kk).reshape(_L, 1)
                    p = lax.gather(
                        v, idx, dn, slice_sizes=(1,),
                        mode=lax.GatherScatterMode.PROMISE_IN_BOUNDS)
                    v = jnp.minimum(v, p)
                return v

            srow = jnp.zeros((_L,), jnp.float32)
            tops = list(tops)
            for t in range(_K + 1):
                mv = tops[0]
                for j in range(1, _K + 1):
                    mv = jnp.minimum(mv, tops[j])
                m = xmin_splat(mv)
                if t > 0:
                    srow = srow + jnp.where(m < jnp.float32(3e38), m, 0.0)
                if t < _K:
                    for j in range(_K + 1):
                        tops[j] = jnp.where(tops[j] == m, jnp.inf, tops[j])
            tops = tuple(tops)
            return tot + srow

        total = lax.fori_loop(0, rows_per_w, row_body, total)

    obuf[...] = total
    pltpu.sync_copy(obuf, out_hbm.at[wid])


def _sc_part(px, py, pz):
    mesh = plsc.VectorSubcoreMesh(core_axis_name="c", subcore_axis_name="s")
    return pl.kernel(
        _sc_body,
        out_type=jax.ShapeDtypeStruct((_NW, _L), jnp.float32),
        mesh=mesh,
        scratch_types=[
            pltpu.VMEM((_N,), jnp.float32),
            pltpu.VMEM((_N,), jnp.float32),
            pltpu.VMEM((_N,), jnp.float32),
            pltpu.VMEM((_L,), jnp.float32),
        ],
    )(px, py, pz)


def kernel(points):
    B, N, D = points.shape
    qp = jnp.pad(points, ((0, 0), (0, 0), (0, _DPAD - D)))
    kt = jnp.transpose(qp, (0, 2, 1))
    px = points[:, :, 0].reshape(-1)
    py = points[:, :, 1].reshape(-1)
    pz = points[:, :, 2].reshape(-1)

    tc_total = _tc_part(qp, kt)
    sc_out = _sc_part(px, py, pz)
    total = tc_total[0, 0] + jnp.sum(sc_out[:, 0])
    return total / (_K * B * N)


# speedup vs baseline: 1.0090x; 1.0090x over previous
import jax
import jax.numpy as jnp
from jax import lax
from jax.experimental import pallas as pl
from jax.experimental.pallas import tpu as pltpu
from jax.experimental.pallas import tpu_sc as plsc

_K = 8
_ROWS = 512
_DPAD = 8
_B = 4
_N = 4096
_SPLIT = 3584
_NW = 32
_L = 16


def _tc_block(q_ref, kt_ref, out_ref):
    b = pl.program_id(0)
    i = pl.program_id(1)

    q = q_ref[0]
    kt = kt_ref[0]

    sq_q = jnp.sum(q * q, axis=1, keepdims=True)
    sq_k = jnp.sum(kt * kt, axis=0, keepdims=True)
    mm = jnp.dot(q * -2.0, kt, preferred_element_type=jnp.float32)
    d2 = mm + sq_q + sq_k
    work = d2.astype(jnp.bfloat16)

    s = jnp.zeros((_ROWS,), dtype=jnp.float32)
    for t in range(_K + 1):
        fold = work
        while fold.shape[1] > 128:
            h = fold.shape[1] // 2
            fold = jnp.minimum(fold[:, :h], fold[:, h:])
        mf = jnp.min(fold.astype(jnp.float32), axis=1)
        if t > 0:
            s = s + jnp.where(jnp.isfinite(mf), mf, 0.0)
        if t < _K:
            m = mf.astype(jnp.bfloat16)
            work = jnp.where(work == m[:, None], jnp.bfloat16(jnp.inf), work)

    partial = jnp.sum(s).reshape(1, 1)

    @pl.when((b == 0) & (i == 0))
    def _init():
        out_ref[:, :] = jnp.zeros((1, 1), dtype=jnp.float32)

    out_ref[:, :] += partial


def _tc_part(qp, kt):
    return pl.pallas_call(
        _tc_block,
        grid=(_B, _SPLIT // _ROWS),
        in_specs=[
            pl.BlockSpec((1, _ROWS, _DPAD), lambda b, i: (b, i, 0)),
            pl.BlockSpec((1, _DPAD, _N), lambda b, i: (b, 0, 0)),
        ],
        out_specs=pl.BlockSpec((1, 1), lambda b, i: (0, 0)),
        out_shape=jax.ShapeDtypeStruct((1, 1), jnp.float32),
    )(qp, kt)


def _sc_body(px_hbm, py_hbm, pz_hbm, out_hbm, px_v, py_v, pz_v, obuf):
    rows_per_w = (_N - _SPLIT) // _NW
    wid = lax.axis_index("s") * 2 + lax.axis_index("c")

    total = jnp.zeros((_L,), jnp.float32)
    for b in range(_B):
        pltpu.sync_copy(px_hbm.at[pl.ds(b * _N, _N)], px_v)
        pltpu.sync_copy(py_hbm.at[pl.ds(b * _N, _N)], py_v)
        pltpu.sync_copy(pz_hbm.at[pl.ds(b * _N, _N)], pz_v)

        def row_body(r, tot):
            row = _SPLIT + wid * rows_per_w + r
            base = (row // _L) * _L
            off = jnp.full((_L, 1), row - base, jnp.int32)
            qsl = pl.ds(base, _L)
            dn = lax.GatherDimensionNumbers(
                offset_dims=(), collapsed_slice_dims=(0,),
                start_index_map=(0,))

            def splat(vec):
                return lax.gather(
                    vec, off, dn, slice_sizes=(1,),
                    mode=lax.GatherScatterMode.PROMISE_IN_BOUNDS)

            qx = splat(px_v[qsl])
            qy = splat(py_v[qsl])
            qz = splat(pz_v[qsl])

            inf16 = jnp.full((_L,), jnp.inf, jnp.float32)
            tops0 = (inf16,) * (_K + 1)

            def chunk_body(c, tops):
                sl = pl.ds(c * _L, _L)
                dx = px_v[sl] - qx
                dy = py_v[sl] - qy
                dz = pz_v[sl] - qz
                x = dx * dx + dy * dy + dz * dz
                new = []
                for j in range(_K + 1):
                    lo = jnp.minimum(tops[j], x)
                    x = jnp.maximum(tops[j], x)
                    new.append(lo)
                return tuple(new)

            tops = lax.fori_loop(0, _N // _L, chunk_body, tops0)

            lanes = lax.iota(jnp.int32, _L)

            def xmin_splat(v):
                for kk in (8, 4, 2, 1):
                    idx = (lanes ^ kk).reshape(_L, 1)
                    p = lax.gather(
                        v, idx, dn, slice_sizes=(1,),
                        mode=lax.GatherScatterMode.PROMISE_IN_BOUNDS)
                    v = jnp.minimum(v, p)
                return v

            srow = jnp.zeros((_L,), jnp.float32)
            tops = list(tops)
            for t in range(_K + 1):
                mv = tops[0]
                for j in range(1, _K + 1):
                    mv = jnp.minimum(mv, tops[j])
                m = xmin_splat(mv)
                if t > 0:
                    srow = srow + jnp.where(m < jnp.float32(3e38), m, 0.0)
                if t < _K:
                    for j in range(_K + 1):
                        tops[j] = jnp.where(tops[j] == m, jnp.inf, tops[j])
            tops = tuple(tops)
            return tot + srow

        total = lax.fori_loop(0, rows_per_w, row_body, total)

    obuf[...] = total
    pltpu.sync_copy(obuf, out_hbm.at[wid])


def _sc_part(px, py, pz):
    mesh = plsc.VectorSubcoreMesh(core_axis_name="c", subcore_axis_name="s")
    return pl.kernel(
        _sc_body,
        out_type=jax.ShapeDtypeStruct((_NW, _L), jnp.float32),
        mesh=mesh,
        scratch_types=[
            pltpu.VMEM((_N,), jnp.float32),
            pltpu.VMEM((_N,), jnp.float32),
            pltpu.VMEM((_N,), jnp.float32),
            pltpu.VMEM((_L,), jnp.float32),
        ],
    )(px, py, pz)


def kernel(points):
    B, N, D = points.shape
    qp = jnp.pad(points, ((0, 0), (0, 0), (0, _DPAD - D)))
    kt = jnp.transpose(qp, (0, 2, 1))
    px = points[:, :, 0].reshape(-1)
    py = points[:, :, 1].reshape(-1)
    pz = points[:, :, 2].reshape(-1)

    tc_total = _tc_part(qp, kt)
    sc_out = _sc_part(px, py, pz)
    total = tc_total[0, 0] + jnp.sum(sc_out[:, 0])
    return total / (_K * B * N)


# speedup vs baseline: 1.0264x; 1.0172x over previous
import jax
import jax.numpy as jnp
from jax import lax
from jax.experimental import pallas as pl
from jax.experimental.pallas import tpu as pltpu
from jax.experimental.pallas import tpu_sc as plsc

_K = 8
_ROWS = 1792
_DPAD = 8
_B = 4
_N = 4096
_SPLIT = 3584
_NW = 32
_L = 16


def _tc_block(q_ref, kt_ref, out_ref):
    b = pl.program_id(0)
    i = pl.program_id(1)

    q = q_ref[0]
    kt = kt_ref[0]

    sq_q = jnp.sum(q * q, axis=1, keepdims=True)
    sq_k = jnp.sum(kt * kt, axis=0, keepdims=True)
    mm = jnp.dot(q * -2.0, kt, preferred_element_type=jnp.float32)
    d2 = mm + sq_q + sq_k
    work = d2.astype(jnp.bfloat16)

    s = jnp.zeros((_ROWS,), dtype=jnp.float32)
    for t in range(_K + 1):
        fold = work
        while fold.shape[1] > 128:
            h = fold.shape[1] // 2
            fold = jnp.minimum(fold[:, :h], fold[:, h:])
        mf = jnp.min(fold.astype(jnp.float32), axis=1)
        if t > 0:
            s = s + jnp.where(jnp.isfinite(mf), mf, 0.0)
        if t < _K:
            m = mf.astype(jnp.bfloat16)
            work = jnp.where(work == m[:, None], jnp.bfloat16(jnp.inf), work)

    partial = jnp.sum(s).reshape(1, 1)

    @pl.when((b == 0) & (i == 0))
    def _init():
        out_ref[:, :] = jnp.zeros((1, 1), dtype=jnp.float32)

    out_ref[:, :] += partial


def _tc_part(qp, kt):
    return pl.pallas_call(
        _tc_block,
        grid=(_B, _SPLIT // _ROWS),
        in_specs=[
            pl.BlockSpec((1, _ROWS, _DPAD), lambda b, i: (b, i, 0)),
            pl.BlockSpec((1, _DPAD, _N), lambda b, i: (b, 0, 0)),
        ],
        out_specs=pl.BlockSpec((1, 1), lambda b, i: (0, 0)),
        out_shape=jax.ShapeDtypeStruct((1, 1), jnp.float32),
    )(qp, kt)


def _sc_body(px_hbm, py_hbm, pz_hbm, out_hbm, px_v, py_v, pz_v, obuf):
    rows_per_w = (_N - _SPLIT) // _NW
    wid = lax.axis_index("s") * 2 + lax.axis_index("c")

    total = jnp.zeros((_L,), jnp.float32)
    for b in range(_B):
        pltpu.sync_copy(px_hbm.at[pl.ds(b * _N, _N)], px_v)
        pltpu.sync_copy(py_hbm.at[pl.ds(b * _N, _N)], py_v)
        pltpu.sync_copy(pz_hbm.at[pl.ds(b * _N, _N)], pz_v)

        def row_body(r, tot):
            row = _SPLIT + wid * rows_per_w + r
            base = (row // _L) * _L
            off = jnp.full((_L, 1), row - base, jnp.int32)
            qsl = pl.ds(base, _L)
            dn = lax.GatherDimensionNumbers(
                offset_dims=(), collapsed_slice_dims=(0,),
                start_index_map=(0,))

            def splat(vec):
                return lax.gather(
                    vec, off, dn, slice_sizes=(1,),
                    mode=lax.GatherScatterMode.PROMISE_IN_BOUNDS)

            qx = splat(px_v[qsl])
            qy = splat(py_v[qsl])
            qz = splat(pz_v[qsl])

            inf16 = jnp.full((_L,), jnp.inf, jnp.float32)
            tops0 = (inf16,) * (_K + 1)

            def chunk_body(c, tops):
                sl = pl.ds(c * _L, _L)
                dx = px_v[sl] - qx
                dy = py_v[sl] - qy
                dz = pz_v[sl] - qz
                x = dx * dx + dy * dy + dz * dz
                new = []
                for j in range(_K + 1):
                    lo = jnp.minimum(tops[j], x)
                    x = jnp.maximum(tops[j], x)
                    new.append(lo)
                return tuple(new)

            tops = lax.fori_loop(0, _N // _L, chunk_body, tops0)

            lanes = lax.iota(jnp.int32, _L)

            def xmin_splat(v):
                for kk in (8, 4, 2, 1):
                    idx = (lanes ^ kk).reshape(_L, 1)
                    p = lax.gather(
                        v, idx, dn, slice_sizes=(1,),
                        mode=lax.GatherScatterMode.PROMISE_IN_BOUNDS)
                    v = jnp.minimum(v, p)
                return v

            srow = jnp.zeros((_L,), jnp.float32)
            tops = list(tops)
            for t in range(_K + 1):
                mv = tops[0]
                for j in range(1, _K + 1):
                    mv = jnp.minimum(mv, tops[j])
                m = xmin_splat(mv)
                if t > 0:
                    srow = srow + jnp.where(m < jnp.float32(3e38), m, 0.0)
                if t < _K:
                    for j in range(_K + 1):
                        tops[j] = jnp.where(tops[j] == m, jnp.inf, tops[j])
            tops = tuple(tops)
            return tot + srow

        total = lax.fori_loop(0, rows_per_w, row_body, total)

    obuf[...] = total
    pltpu.sync_copy(obuf, out_hbm.at[wid])


def _sc_part(px, py, pz):
    mesh = plsc.VectorSubcoreMesh(core_axis_name="c", subcore_axis_name="s")
    return pl.kernel(
        _sc_body,
        out_type=jax.ShapeDtypeStruct((_NW, _L), jnp.float32),
        mesh=mesh,
        scratch_types=[
            pltpu.VMEM((_N,), jnp.float32),
            pltpu.VMEM((_N,), jnp.float32),
            pltpu.VMEM((_N,), jnp.float32),
            pltpu.VMEM((_L,), jnp.float32),
        ],
    )(px, py, pz)


def kernel(points):
    B, N, D = points.shape
    qp = jnp.pad(points, ((0, 0), (0, 0), (0, _DPAD - D)))
    kt = jnp.transpose(qp, (0, 2, 1))
    px = points[:, :, 0].reshape(-1)
    py = points[:, :, 1].reshape(-1)
    pz = points[:, :, 2].reshape(-1)

    tc_total = _tc_part(qp, kt)
    sc_out = _sc_part(px, py, pz)
    total = tc_total[0, 0] + jnp.sum(sc_out[:, 0])
    return total / (_K * B * N)
